# baseline (speedup 1.0000x reference)
import jax
import jax.numpy as jnp
from jax import lax
from jax.experimental import pallas as pl
from jax.experimental.pallas import tpu as pltpu
from jax.experimental.pallas import tpu_sc as plsc

NNODE = 10000
NEDGE = 320000
NPAD = 10240
CH = 128
NCHP = 2560
EPAD = NCHP * CH
NC = 2
NS = 16
SLICE = NPAD // NS
KPT = NCHP // (NC * NS)
DSEG = 32
DEGPT = NCHP // NS
ASEGS = (32, 32, 16)

f32 = jnp.float32
i32 = jnp.int32


def _rsqrt16(x):
    xi = plsc.bitcast(x, i32)
    yi = jnp.full((16,), 0x5F3759DF, i32) - lax.shift_right_logical(
        xi, jnp.ones((16,), i32))
    y = plsc.bitcast(yi, f32)
    for _ in range(3):
        y = y * (1.5 - 0.5 * x * y * y)
    return y


def _fill16(v):
    return jnp.full((16,), v, i32)


def _agg_pipeline(nk, src_all, dst_all, w_all, rows, feat_hbm, acc_sh,
                  sems, nvec):
    sg = (sems[0], sems[1])
    sc = (sems[2], sems[3])

    def g_start(k, b):
        pltpu.async_copy(feat_hbm.at[src_all.at[k]], rows.at[b], sg[b])

    def g_wait(k, b):
        pltpu.make_async_copy(feat_hbm.at[src_all.at[k]], rows.at[b],
                              sg[b]).wait()

    def c_start(k, b):
        pltpu.async_copy(rows.at[b], acc_sh.at[dst_all.at[k]], sc[b],
                         add=True)

    def c_wait(k, b):
        pltpu.make_async_copy(rows.at[b], acc_sh.at[dst_all.at[k]],
                              sc[b]).wait()

    def scale(k, b):
        @pl.loop(0, CH)
        def _(r):
            sp = plsc.load_gather(w_all, [_fill16(k), _fill16(r)])
            for j in range(nvec):
                rows[b, r, pl.ds(j * 16, 16)] = (
                    rows[b, r, pl.ds(j * 16, 16)] * sp)

    g_start(0, 0)

    @pl.loop(0, nk // 2)
    def _(g):
        k0 = g * 2
        @pl.when(g > 0)
        def _():
            c_wait(k0, 1)
        g_start(k0 + 1, 1)
        g_wait(k0, 0)
        scale(k0, 0)
        c_start(k0, 0)
        @pl.when(g < nk // 2 - 1)
        def _():
            c_wait(k0, 0)
            g_start(k0 + 2, 0)
        g_wait(k0 + 1, 1)
        scale(k0 + 1, 1)
        c_start(k0 + 1, 1)

    c_wait(0, 0)
    c_wait(0, 1)


def _l1_body(src_hbm, dst_hbm, w_hbm, x_hbm,
             agg_hbm, dinv_hbm, xs_hbm,
             src_all, dst_all, w_all, rows, idx80, dbuf,
             deg_sh, acc_sh, sg0, sg1, sc0, sc1):
    c = lax.axis_index("c")
    s = lax.axis_index("s")
    z16 = jnp.zeros((16,), f32)
    c7 = jnp.full((16,), 7, i32)
    c127 = jnp.full((16,), 127, i32)
    nrow = NPAD // CH // NS

    @pl.loop(0, CH)
    def _(r):
        for j in range(8):
            rows[0, r, pl.ds(j * 16, 16)] = z16

    for m in range(NPAD // CH // 16):
        idx80[pl.ds(m * 16, 16)] = lax.iota(i32, 16) + m * 16

    for m in range(SLICE // CH):
        pltpu.sync_copy(rows.at[0], acc_sh.at[pl.ds(s * SLICE + m * CH, CH), :])
    pltpu.sync_copy(rows.at[0, pl.ds(0, nrow), :],
                    deg_sh.at[pl.ds(s * nrow, nrow), :])

    plsc.subcore_barrier()

    ks0 = s * DEGPT
    for t in range(DEGPT // DSEG):
        pltpu.sync_copy(dst_hbm.at[pl.ds(ks0 + t * DSEG, DSEG), :], dst_all)
        pltpu.sync_copy(w_hbm.at[pl.ds(ks0 + t * DSEG, DSEG), :], w_all)

        @pl.loop(0, DSEG * (CH // 16))
        def _(g):
            d16 = dst_all[g // 8, pl.ds((g % 8) * 16, 16)]
            w16 = w_all[g // 8, pl.ds((g % 8) * 16, 16)]
            plsc.addupdate_scatter(
                rows.at[0],
                [lax.shift_right_logical(d16, c7),
                 jnp.bitwise_and(d16, c127)],
                w16)

    pltpu.sync_copy(rows.at[0, pl.ds(0, NPAD // CH), :],
                    deg_sh.at[idx80], add=True)
    plsc.subcore_barrier()

    pltpu.sync_copy(deg_sh.at[pl.ds(s * nrow, nrow), :], dbuf)
    for r in range(nrow):
        for j in range(8):
            d = dbuf[r, pl.ds(j * 16, 16)]
            dbuf[r, pl.ds(j * 16, 16)] = _rsqrt16(d + 1.0)

    @pl.when(c == 0)
    def _():
        for r in range(nrow):
            pltpu.sync_copy(dbuf.at[r],
                            dinv_hbm.at[pl.ds(s * SLICE + r * CH, CH)])

    for m in range(SLICE // CH):
        pltpu.sync_copy(x_hbm.at[pl.ds(s * SLICE + m * CH, CH), :], rows.at[0])

        @pl.loop(0, CH)
        def _(r):
            sp = plsc.load_gather(dbuf, [_fill16(m), _fill16(r)])
            for j in range(8):
                rows[0, r, pl.ds(j * 16, 16)] = (
                    rows[0, r, pl.ds(j * 16, 16)] * sp)
        pltpu.sync_copy(rows.at[0], xs_hbm.at[pl.ds(s * SLICE + m * CH, CH), :])

    plsc.subcore_barrier()

    ka = c * (NCHP // NC) + s * KPT
    for t, nk in enumerate(ASEGS):
        seg0 = ka + sum(ASEGS[:t])
        pltpu.sync_copy(src_hbm.at[pl.ds(seg0, nk), :],
                        src_all.at[pl.ds(0, nk), :])
        pltpu.sync_copy(dst_hbm.at[pl.ds(seg0, nk), :],
                        dst_all.at[pl.ds(0, nk), :])
        pltpu.sync_copy(w_hbm.at[pl.ds(seg0, nk), :],
                        w_all.at[pl.ds(0, nk), :])
        _agg_pipeline(nk, src_all, dst_all, w_all, rows, xs_hbm, acc_sh,
                      (sg0, sg1, sc0, sc1), 8)

    plsc.subcore_barrier()
    pltpu.sync_copy(acc_sh.at[pl.ds(s * SLICE, SLICE), :],
                    agg_hbm.at[c, pl.ds(s * SLICE, SLICE), :])


def _sc_layer1(src, dst, w, x):
    mesh = plsc.VectorSubcoreMesh(core_axis_name="c", subcore_axis_name="s",
                                  num_cores=NC, num_subcores=NS)
    return pl.kernel(
        _l1_body,
        out_type=(jax.ShapeDtypeStruct((NC, NPAD, 128), f32),
                  jax.ShapeDtypeStruct((NPAD,), f32),
                  jax.ShapeDtypeStruct((NPAD, 128), f32)),
        mesh=mesh,
        scratch_types=[
            pltpu.VMEM((32, CH), i32),
            pltpu.VMEM((32, CH), i32),
            pltpu.VMEM((32, CH), f32),
            pltpu.VMEM((2, CH, 128), f32),
            pltpu.VMEM((NPAD // CH,), i32),
            pltpu.VMEM((NPAD // CH // NS, CH), f32),
            pltpu.VMEM_SHARED((NPAD // CH, CH), f32),
            pltpu.VMEM_SHARED((NPAD, 128), f32),
            pltpu.SemaphoreType.DMA,
            pltpu.SemaphoreType.DMA,
            pltpu.SemaphoreType.DMA,
            pltpu.SemaphoreType.DMA,
        ],
        compiler_params=pltpu.CompilerParams(needs_layout_passes=False),
        name="gcn_sc_layer1",
    )(src, dst, w, x)


def _l2_body(src_hbm, dst_hbm, w_hbm, hws_hbm, agg_hbm,
             src_all, dst_all, w_all, rows, acc_sh, sg0, sg1, sc0, sc1):
    c = lax.axis_index("c")
    s = lax.axis_index("s")
    z16 = jnp.zeros((16,), f32)

    @pl.loop(0, CH)
    def _(r):
        rows[0, r, pl.ds(0, 16)] = z16
        rows[0, r, pl.ds(16, 16)] = z16
    for m in range(SLICE // CH):
        pltpu.sync_copy(rows.at[0], acc_sh.at[pl.ds(s * SLICE + m * CH, CH), :])
    plsc.subcore_barrier()

    ka = c * (NCHP // NC) + s * KPT
    pltpu.sync_copy(src_hbm.at[pl.ds(ka, KPT), :], src_all)
    pltpu.sync_copy(dst_hbm.at[pl.ds(ka, KPT), :], dst_all)
    pltpu.sync_copy(w_hbm.at[pl.ds(ka, KPT), :], w_all)
    _agg_pipeline(KPT, src_all, dst_all, w_all, rows, hws_hbm, acc_sh,
                  (sg0, sg1, sc0, sc1), 2)

    plsc.subcore_barrier()
    pltpu.sync_copy(acc_sh.at[pl.ds(s * SLICE, SLICE), :],
                    agg_hbm.at[c, pl.ds(s * SLICE, SLICE), :])


def _sc_layer2(src, dst, w, hws):
    mesh = plsc.VectorSubcoreMesh(core_axis_name="c", subcore_axis_name="s",
                                  num_cores=NC, num_subcores=NS)
    return pl.kernel(
        _l2_body,
        out_type=jax.ShapeDtypeStruct((NC, NPAD, 32), f32),
        mesh=mesh,
        scratch_types=[
            pltpu.VMEM((KPT, CH), i32),
            pltpu.VMEM((KPT, CH), i32),
            pltpu.VMEM((KPT, CH), f32),
            pltpu.VMEM((2, CH, 32), f32),
            pltpu.VMEM_SHARED((NPAD, 32), f32),
            pltpu.SemaphoreType.DMA,
            pltpu.SemaphoreType.DMA,
            pltpu.SemaphoreType.DMA,
            pltpu.SemaphoreType.DMA,
        ],
        compiler_params=pltpu.CompilerParams(needs_layout_passes=False,
                                             use_tc_tiling_on_sc=False),
        name="gcn_sc_layer2",
    )(src, dst, w, hws)


BM = 200


def _tc_mid_body(a0, a1, xr, dv, w1, b1, w2, hw_out, hws_out):
    d = dv[...]
    z = (a0[...] + a1[...]) * d + xr[...] * (d * d)
    h = jnp.dot(z, w1[...], preferred_element_type=f32) + b1[...]
    h = jnp.maximum(h, 0.0)
    hw = jnp.dot(h, w2[...], preferred_element_type=f32)
    hw_out[...] = hw
    hws_out[...] = hw * d


def _tc_mid(a0, a1, x, dinv_col, W1p, b1p, W2p):
    return pl.pallas_call(
        _tc_mid_body,
        grid=(NNODE // BM,),
        in_specs=[
            pl.BlockSpec((BM, 128), lambda i: (i, 0)),
            pl.BlockSpec((BM, 128), lambda i: (i, 0)),
            pl.BlockSpec((BM, 128), lambda i: (i, 0)),
            pl.BlockSpec((BM, 1), lambda i: (i, 0)),
            pl.BlockSpec((128, 256), lambda i: (0, 0)),
            pl.BlockSpec((1, 256), lambda i: (0, 0)),
            pl.BlockSpec((256, 32), lambda i: (0, 0)),
        ],
        out_specs=[
            pl.BlockSpec((BM, 32), lambda i: (i, 0)),
            pl.BlockSpec((BM, 32), lambda i: (i, 0)),
        ],
        out_shape=[
            jax.ShapeDtypeStruct((NNODE, 32), f32),
            jax.ShapeDtypeStruct((NNODE, 32), f32),
        ],
        name="gcn_tc_mid",
    )(a0, a1, x, dinv_col, W1p, b1p, W2p)


def _tc_fin_body(q0, q1, hwr, dv, b2, out):
    d = dv[...]
    out[...] = (q0[...] + q1[...]) * d + hwr[...] * (d * d) + b2[...]


def _tc_fin(q0, q1, hw, dinv_col, b2p):
    return pl.pallas_call(
        _tc_fin_body,
        grid=(NNODE // BM,),
        in_specs=[
            pl.BlockSpec((BM, 32), lambda i: (i, 0)),
            pl.BlockSpec((BM, 32), lambda i: (i, 0)),
            pl.BlockSpec((BM, 32), lambda i: (i, 0)),
            pl.BlockSpec((BM, 1), lambda i: (i, 0)),
            pl.BlockSpec((1, 32), lambda i: (0, 0)),
        ],
        out_specs=pl.BlockSpec((BM, 32), lambda i: (i, 0)),
        out_shape=jax.ShapeDtypeStruct((NNODE, 32), f32),
        name="gcn_tc_fin",
    )(q0, q1, hw, dinv_col, b2p)


def kernel(x, edge_index, edge_weight, W1, b1, W2, b2):
    src = jnp.pad(edge_index[0], (0, EPAD - NEDGE)).reshape(NCHP, CH)
    dst = jnp.pad(edge_index[1], (0, EPAD - NEDGE)).reshape(NCHP, CH)
    w = jnp.pad(edge_weight, (0, EPAD - NEDGE)).reshape(NCHP, CH)
    xp = jnp.pad(x, ((0, NPAD - NNODE), (0, 0)))
    agg1, dinv1d, _ = _sc_layer1(src, dst, w, xp)
    dinv_col = dinv1d[:NNODE].reshape(NNODE, 1)
    W1p = jnp.pad(W1, ((0, 0), (0, 56)))
    b1p = jnp.pad(b1, (0, 56)).reshape(1, 256)
    W2p = jnp.pad(W2, ((0, 56), (0, 12)))
    hw, hws = _tc_mid(agg1[0, :NNODE], agg1[1, :NNODE], x, dinv_col,
                      W1p, b1p, W2p)
    agg2 = _sc_layer2(src, dst, w, hws)
    b2p = jnp.pad(b2, (0, 12)).reshape(1, 32)
    out = _tc_fin(agg2[0, :NNODE], agg2[1, :NNODE], hw, dinv_col, b2p)
    return out[:, :20]


# speedup vs baseline: 1.0691x; 1.0691x over previous
import jax
import jax.numpy as jnp
from jax import lax
from jax.experimental import pallas as pl
from jax.experimental.pallas import tpu as pltpu
from jax.experimental.pallas import tpu_sc as plsc

NNODE = 10000
NEDGE = 320000
NPAD = 10240
CH = 64
NCHP = 5120
EPAD = NCHP * CH
NC = 2
NS = 16
SLICE = NPAD // NS
KPT = NCHP // (NC * NS)
DSEG = 64
DEGPT = NCHP // NS
ASEGS = (64, 64, 32)
NBUF = 4

f32 = jnp.float32
i32 = jnp.int32


def _rsqrt16(x):
    xi = plsc.bitcast(x, i32)
    yi = jnp.full((16,), 0x5F3759DF, i32) - lax.shift_right_logical(
        xi, jnp.ones((16,), i32))
    y = plsc.bitcast(yi, f32)
    for _ in range(3):
        y = y * (1.5 - 0.5 * x * y * y)
    return y


def _fill16(v):
    return jnp.full((16,), v, i32)


def _agg_pipeline(nk, src_all, dst_all, w_all, rows, feat_hbm, acc_sh,
                  sg, sc, nvec):
    def g_start(k, b):
        pltpu.async_copy(feat_hbm.at[src_all.at[k]], rows.at[b], sg[b])

    def g_wait(k, b):
        pltpu.make_async_copy(feat_hbm.at[src_all.at[k]], rows.at[b],
                              sg[b]).wait()

    def c_start(k, b):
        pltpu.async_copy(rows.at[b], acc_sh.at[dst_all.at[k]], sc[b],
                         add=True)

    def c_wait(k, b):
        pltpu.make_async_copy(rows.at[b], acc_sh.at[dst_all.at[k]],
                              sc[b]).wait()

    def scale(k, b):
        @pl.loop(0, CH)
        def _(r):
            sp = plsc.load_gather(w_all, [_fill16(k), _fill16(r)])
            for j in range(nvec):
                rows[b, r, pl.ds(j * 16, 16)] = (
                    rows[b, r, pl.ds(j * 16, 16)] * sp)

    g_start(0, 0)
    g_start(1, 1)

    @pl.loop(0, nk // NBUF)
    def _(g):
        k0 = g * NBUF
        for b in range(NBUF):
            k = k0 + b
            bp = (b + 2) % NBUF
            @pl.when(k + 2 < nk)
            def _():
                @pl.when(k >= 2)
                def _():
                    c_wait(k, bp)
                g_start(k + 2, bp)

            g_wait(k, b)
            scale(k, b)
            c_start(k, b)

    for b in range(NBUF):
        c_wait(nk - NBUF + b, b)


def _l1_body(src_hbm, dst_hbm, w_hbm, x_hbm,
             agg_hbm, dinv_hbm, xs_hbm,
             src_all, dst_all, w_all, rows, idx128, dbuf,
             deg_sh, acc_sh,
             sg0, sg1, sg2, sg3, sc0, sc1, sc2, sc3):
    c = lax.axis_index("c")
    s = lax.axis_index("s")
    sg = (sg0, sg1, sg2, sg3)
    sc = (sc0, sc1, sc2, sc3)
    z16 = jnp.zeros((16,), f32)
    c7 = jnp.full((16,), 7, i32)
    c13 = jnp.full((16,), 13, i32)
    c63 = jnp.full((16,), 63, i32)
    c127 = jnp.full((16,), 127, i32)
    nrow = NPAD // CH // NS

    @pl.loop(0, CH)
    def _(r):
        for b in range(2):
            for j in range(8):
                rows[b, r, pl.ds(j * 16, 16)] = z16

    for m in range(8):
        idx128[m // 4, pl.ds((m % 4) * 16, 16)] = lax.iota(i32, 16) + m * 16

    for m in range(SLICE // CH):
        pltpu.sync_copy(rows.at[0], acc_sh.at[pl.ds(s * SLICE + m * CH, CH), :])
    pltpu.sync_copy(rows.at[0, pl.ds(0, 8), :],
                    deg_sh.at[pl.ds(s * 8, 8), :])

    plsc.subcore_barrier()

    ks0 = s * DEGPT
    for t in range(DEGPT // DSEG):
        pltpu.sync_copy(dst_hbm.at[pl.ds(ks0 + t * DSEG, DSEG), :], dst_all)
        pltpu.sync_copy(w_hbm.at[pl.ds(ks0 + t * DSEG, DSEG), :], w_all)

        @pl.loop(0, DSEG * (CH // 16))
        def _(g):
            d16 = dst_all[g // 4, pl.ds((g % 4) * 16, 16)]
            w16 = w_all[g // 4, pl.ds((g % 4) * 16, 16)]
            plsc.addupdate_scatter(
                rows,
                [lax.shift_right_logical(d16, c13),
                 jnp.bitwise_and(lax.shift_right_logical(d16, c7), c63),
                 jnp.bitwise_and(d16, c127)],
                w16)

    pltpu.sync_copy(rows.at[0], deg_sh.at[idx128.at[0]], add=True)
    pltpu.sync_copy(rows.at[1], deg_sh.at[idx128.at[1]], add=True)
    plsc.subcore_barrier()

    pltpu.sync_copy(deg_sh.at[pl.ds(s * 5, 5), :], dbuf)
    for r in range(5):
        for j in range(8):
            d = dbuf[r, pl.ds(j * 16, 16)]
            dbuf[r, pl.ds(j * 16, 16)] = _rsqrt16(d + 1.0)

    @pl.when(c == 0)
    def _():
        for r in range(5):
            pltpu.sync_copy(dbuf.at[r],
                            dinv_hbm.at[pl.ds(s * SLICE + r * 128, 128)])

    for m in range(SLICE // CH):
        pltpu.sync_copy(x_hbm.at[pl.ds(s * SLICE + m * CH, CH), :], rows.at[0])

        @pl.loop(0, CH)
        def _(r):
            q = m * CH + r
            sp = plsc.load_gather(dbuf, [_fill16(q // 128), _fill16(q % 128)])
            for j in range(8):
                rows[0, r, pl.ds(j * 16, 16)] = (
                    rows[0, r, pl.ds(j * 16, 16)] * sp)
        pltpu.sync_copy(rows.at[0], xs_hbm.at[pl.ds(s * SLICE + m * CH, CH), :])

    plsc.subcore_barrier()

    ka = c * (NCHP // NC) + s * KPT
    for t, nk in enumerate(ASEGS):
        seg0 = ka + sum(ASEGS[:t])
        pltpu.sync_copy(src_hbm.at[pl.ds(seg0, nk), :],
                        src_all.at[pl.ds(0, nk), :])
        pltpu.sync_copy(dst_hbm.at[pl.ds(seg0, nk), :],
                        dst_all.at[pl.ds(0, nk), :])
        pltpu.sync_copy(w_hbm.at[pl.ds(seg0, nk), :],
                        w_all.at[pl.ds(0, nk), :])
        _agg_pipeline(nk, src_all, dst_all, w_all, rows, xs_hbm, acc_sh,
                      sg, sc, 8)

    plsc.subcore_barrier()
    pltpu.sync_copy(acc_sh.at[pl.ds(s * SLICE, SLICE), :],
                    agg_hbm.at[c, pl.ds(s * SLICE, SLICE), :])


def _sc_layer1(src, dst, w, x):
    mesh = plsc.VectorSubcoreMesh(core_axis_name="c", subcore_axis_name="s",
                                  num_cores=NC, num_subcores=NS)
    return pl.kernel(
        _l1_body,
        out_type=(jax.ShapeDtypeStruct((NC, NPAD, 128), f32),
                  jax.ShapeDtypeStruct((NPAD,), f32),
                  jax.ShapeDtypeStruct((NPAD, 128), f32)),
        mesh=mesh,
        scratch_types=[
            pltpu.VMEM((DSEG, CH), i32),
            pltpu.VMEM((DSEG, CH), i32),
            pltpu.VMEM((DSEG, CH), f32),
            pltpu.VMEM((NBUF, CH, 128), f32),
            pltpu.VMEM((2, CH), i32),
            pltpu.VMEM((5, 128), f32),
            pltpu.VMEM_SHARED((128, 128), f32),
            pltpu.VMEM_SHARED((NPAD, 128), f32),
            pltpu.SemaphoreType.DMA, pltpu.SemaphoreType.DMA,
            pltpu.SemaphoreType.DMA, pltpu.SemaphoreType.DMA,
            pltpu.SemaphoreType.DMA, pltpu.SemaphoreType.DMA,
            pltpu.SemaphoreType.DMA, pltpu.SemaphoreType.DMA,
        ],
        compiler_params=pltpu.CompilerParams(needs_layout_passes=False,
                                             use_tc_tiling_on_sc=False),
        name="gcn_sc_layer1",
    )(src, dst, w, x)


def _l2_body(src_hbm, dst_hbm, w_hbm, hws_hbm, agg_hbm,
             src_all, dst_all, w_all, rows, acc_sh,
             sg0, sg1, sg2, sg3, sc0, sc1, sc2, sc3):
    c = lax.axis_index("c")
    s = lax.axis_index("s")
    z16 = jnp.zeros((16,), f32)

    @pl.loop(0, CH)
    def _(r):
        rows[0, r, pl.ds(0, 16)] = z16
        rows[0, r, pl.ds(16, 16)] = z16
    for m in range(SLICE // CH):
        pltpu.sync_copy(rows.at[0], acc_sh.at[pl.ds(s * SLICE + m * CH, CH), :])
    plsc.subcore_barrier()

    ka = c * (NCHP // NC) + s * KPT
    pltpu.sync_copy(src_hbm.at[pl.ds(ka, KPT), :], src_all)
    pltpu.sync_copy(dst_hbm.at[pl.ds(ka, KPT), :], dst_all)
    pltpu.sync_copy(w_hbm.at[pl.ds(ka, KPT), :], w_all)
    _agg_pipeline(KPT, src_all, dst_all, w_all, rows, hws_hbm, acc_sh,
                  (sg0, sg1, sg2, sg3), (sc0, sc1, sc2, sc3), 2)

    plsc.subcore_barrier()
    pltpu.sync_copy(acc_sh.at[pl.ds(s * SLICE, SLICE), :],
                    agg_hbm.at[c, pl.ds(s * SLICE, SLICE), :])


def _sc_layer2(src, dst, w, hws):
    mesh = plsc.VectorSubcoreMesh(core_axis_name="c", subcore_axis_name="s",
                                  num_cores=NC, num_subcores=NS)
    return pl.kernel(
        _l2_body,
        out_type=jax.ShapeDtypeStruct((NC, NPAD, 32), f32),
        mesh=mesh,
        scratch_types=[
            pltpu.VMEM((KPT, CH), i32),
            pltpu.VMEM((KPT, CH), i32),
            pltpu.VMEM((KPT, CH), f32),
            pltpu.VMEM((NBUF, CH, 32), f32),
            pltpu.VMEM_SHARED((NPAD, 32), f32),
            pltpu.SemaphoreType.DMA, pltpu.SemaphoreType.DMA,
            pltpu.SemaphoreType.DMA, pltpu.SemaphoreType.DMA,
            pltpu.SemaphoreType.DMA, pltpu.SemaphoreType.DMA,
            pltpu.SemaphoreType.DMA, pltpu.SemaphoreType.DMA,
        ],
        compiler_params=pltpu.CompilerParams(needs_layout_passes=False,
                                             use_tc_tiling_on_sc=False),
        name="gcn_sc_layer2",
    )(src, dst, w, hws)


BM = 200


def _tc_mid_body(a0, a1, xr, dv, w1, b1, w2, hw_out, hws_out):
    d = dv[...]
    z = (a0[...] + a1[...]) * d + xr[...] * (d * d)
    h = jnp.dot(z, w1[...], preferred_element_type=f32) + b1[...]
    h = jnp.maximum(h, 0.0)
    hw = jnp.dot(h, w2[...], preferred_element_type=f32)
    hw_out[...] = hw
    hws_out[...] = hw * d


def _tc_mid(a0, a1, x, dinv_col, W1p, b1p, W2p):
    return pl.pallas_call(
        _tc_mid_body,
        grid=(NNODE // BM,),
        in_specs=[
            pl.BlockSpec((BM, 128), lambda i: (i, 0)),
            pl.BlockSpec((BM, 128), lambda i: (i, 0)),
            pl.BlockSpec((BM, 128), lambda i: (i, 0)),
            pl.BlockSpec((BM, 1), lambda i: (i, 0)),
            pl.BlockSpec((128, 256), lambda i: (0, 0)),
            pl.BlockSpec((1, 256), lambda i: (0, 0)),
            pl.BlockSpec((256, 32), lambda i: (0, 0)),
        ],
        out_specs=[
            pl.BlockSpec((BM, 32), lambda i: (i, 0)),
            pl.BlockSpec((BM, 32), lambda i: (i, 0)),
        ],
        out_shape=[
            jax.ShapeDtypeStruct((NNODE, 32), f32),
            jax.ShapeDtypeStruct((NNODE, 32), f32),
        ],
        name="gcn_tc_mid",
    )(a0, a1, x, dinv_col, W1p, b1p, W2p)


def _tc_fin_body(q0, q1, hwr, dv, b2, out):
    d = dv[...]
    out[...] = (q0[...] + q1[...]) * d + hwr[...] * (d * d) + b2[...]


def _tc_fin(q0, q1, hw, dinv_col, b2p):
    return pl.pallas_call(
        _tc_fin_body,
        grid=(NNODE // BM,),
        in_specs=[
            pl.BlockSpec((BM, 32), lambda i: (i, 0)),
            pl.BlockSpec((BM, 32), lambda i: (i, 0)),
            pl.BlockSpec((BM, 32), lambda i: (i, 0)),
            pl.BlockSpec((BM, 1), lambda i: (i, 0)),
            pl.BlockSpec((1, 32), lambda i: (0, 0)),
        ],
        out_specs=pl.BlockSpec((BM, 32), lambda i: (i, 0)),
        out_shape=jax.ShapeDtypeStruct((NNODE, 32), f32),
        name="gcn_tc_fin",
    )(q0, q1, hw, dinv_col, b2p)


def kernel(x, edge_index, edge_weight, W1, b1, W2, b2):
    src = jnp.pad(edge_index[0], (0, EPAD - NEDGE)).reshape(NCHP, CH)
    dst = jnp.pad(edge_index[1], (0, EPAD - NEDGE)).reshape(NCHP, CH)
    w = jnp.pad(edge_weight, (0, EPAD - NEDGE)).reshape(NCHP, CH)
    xp = jnp.pad(x, ((0, NPAD - NNODE), (0, 0)))
    agg1, dinv1d, _ = _sc_layer1(src, dst, w, xp)
    dinv_col = dinv1d[:NNODE].reshape(NNODE, 1)
    W1p = jnp.pad(W1, ((0, 0), (0, 56)))
    b1p = jnp.pad(b1, (0, 56)).reshape(1, 256)
    W2p = jnp.pad(W2, ((0, 56), (0, 12)))
    hw, hws = _tc_mid(agg1[0, :NNODE], agg1[1, :NNODE], x, dinv_col,
                      W1p, b1p, W2p)
    agg2 = _sc_layer2(src, dst, w, hws)
    b2p = jnp.pad(b2, (0, 12)).reshape(1, 32)
    out = _tc_fin(agg2[0, :NNODE], agg2[1, :NNODE], hw, dinv_col, b2p)
    return out[:, :20]
